# baseline (speedup 1.0000x reference)
import jax
import jax.numpy as jnp
import numpy as np
from jax import lax
from jax.experimental import pallas as pl
from jax.experimental.pallas import tpu as pltpu

_NS = 18
_B = 4096
_D = 256
_O = 6
_R = 2 * _O * _NS
_BLK = 2048

_ROT = ((13, 15, 26, 6), (17, 29, 16, 24))
_TINY = np.float32(np.finfo(np.float32).tiny)


def _key_cols():
    keys = jnp.concatenate(
        [jax.random.split(jax.random.fold_in(jax.random.key(42), op_i), _NS)
         for op_i in range(2)])
    kd = jax.random.key_data(keys).reshape(2, _NS, 2)
    k1 = jnp.broadcast_to(kd[:, None, :, 0], (2, _O, _NS)).reshape(_R)
    k2 = jnp.broadcast_to(kd[:, None, :, 1], (2, _O, _NS)).reshape(_R)
    cr = ((np.arange(_R) // _NS) % _O).astype(np.uint32)
    ks2 = k1 ^ k2 ^ np.uint32(0x1BD11BDA)
    return jnp.stack([k1, k2, ks2, ks2 + 1, k1 + 2, k2 + 3, ks2 + 4,
                      k1 + 5, k2 + cr], axis=1)


def _rotl(x, d):
    return lax.shift_left(x, np.uint32(d)) | lax.shift_right_logical(
        x, np.uint32(32 - d))


def _gen_gumbel(karr_ref, blk_base, blk, g_s):
    c = [karr_ref[:, j:j + 1] for j in range(9)]
    lanes = lax.broadcasted_iota(jnp.uint32, (1, blk), 1)
    x1 = c[8] + np.uint32(6) * (lanes + blk_base)
    x0 = jnp.broadcast_to(c[0], (_R, blk)) + jnp.zeros((_R, blk), jnp.uint32)
    inj = ((c[1], c[3]), (c[2], c[4]), (c[0], c[5]), (c[1], c[6]),
           (c[2], c[7]))
    for g in range(5):
        for r in _ROT[g % 2]:
            x0 = x0 + x1
            x1 = _rotl(x1, r)
            x1 = x1 ^ x0
        a, b = inj[g]
        x0 = x0 + a
        x1 = x1 + b
    bits = x0 ^ x1
    fb = lax.shift_right_logical(bits, np.uint32(9)) | np.uint32(0x3F800000)
    f = lax.bitcast_convert_type(fb, jnp.float32) - jnp.float32(1.0)
    u = lax.max(jnp.float32(_TINY), f + jnp.float32(_TINY))
    g_s[...] = -jnp.log(-jnp.log(u))


def _fwd_kernel(x_ref, xh_ref, karr_ref, w1_ref, b1_ref, w2_ref, b2_ref,
                w_ref, b_ref, out_ref, g_s):
    blk = x_ref.shape[0]
    blk_base = (pl.program_id(0) * blk).astype(jnp.uint32)
    _gen_gumbel(karr_ref, blk_base, blk, g_s)

    dn = (((1,), (1,)), ((), ()))
    wfull = w_ref[...]
    wx = wfull[:, :_D]
    xb = x_ref[...]
    xh = xh_ref[...]

    idxs = []
    tvecs = []
    for o, (wr, br) in enumerate(((w1_ref, b1_ref), (w2_ref, b2_ref))):
        wo = wr[...]
        bo = br[...]
        z = jax.lax.dot_general(wo, xb, dn,
                                preferred_element_type=jnp.float32) + bo
        m = jnp.max(z, axis=0, keepdims=True)
        e = jnp.exp(z - m)
        dist = e / jnp.sum(e, axis=0, keepdims=True)
        logits = jnp.log(dist)

        zh = jax.lax.dot_general(wo, xh, dn,
                                 preferred_element_type=jnp.float32) + bo
        mh = jnp.max(zh, axis=0, keepdims=True)
        eh = jnp.exp(zh - mh)
        disth = eh / jnp.sum(eh, axis=0, keepdims=True)
        tvecs.append(disth[0, :])

        best = None
        idx = None
        for cc in range(6):
            r = (o * 6 + cc) * _NS
            val = logits[cc:cc + 1, :] + g_s[r:r + _NS, :]
            if cc == 0:
                best = val
                idx = jnp.zeros_like(val)
            else:
                pred = val > best
                best = jnp.where(pred, val, best)
                idx = jnp.where(pred, jnp.float32(cc), idx)
        idxs.append(idx)

    y0 = jax.lax.dot_general(wx, xb, dn,
                             preferred_element_type=jnp.float32) + b_ref[...]

    accs = [jnp.broadcast_to(y0[j:j + 1, :], (_NS, blk)) for j in range(6)]
    ps = []
    for o in range(2):
        p = None
        for cc in range(6):
            mf = (idxs[o] == jnp.float32(cc)).astype(jnp.float32)
            tc = tvecs[o][cc]
            p = mf * tc if p is None else p + mf * tc
            for j in range(6):
                accs[j] = accs[j] + mf * wfull[j, _D + 6 * o + cc]
        ps.append(p)

    m = accs[0]
    for j in range(1, 6):
        m = jnp.maximum(m, accs[j])
    es = [jnp.exp(a - m) for a in accs]
    se = es[0]
    for j in range(1, 6):
        se = se + es[j]

    w = ps[0] * ps[1]
    u = w / se
    norm = jnp.sum(w, axis=0, keepdims=True)
    rows = [jnp.sum(u * es[j], axis=0, keepdims=True) / norm
            for j in range(6)]
    out_ref[...] = jnp.concatenate(rows, axis=0).T


def kernel(x, W_opp1, b_opp1, W_opp2, b_opp2, W, b):
    karr = _key_cols()
    b1 = b_opp1.reshape(_O, 1)
    b2 = b_opp2.reshape(_O, 1)
    br = b.reshape(_O, 1)

    out = pl.pallas_call(
        _fwd_kernel,
        grid=(_B // _BLK,),
        in_specs=[
            pl.BlockSpec((_BLK, _D), lambda i: (i, 0)),
            pl.BlockSpec((8, _D), lambda i: (0, 0)),
            pl.BlockSpec((_R, 9), lambda i: (0, 0)),
            pl.BlockSpec((_O, _D), lambda i: (0, 0)),
            pl.BlockSpec((_O, 1), lambda i: (0, 0)),
            pl.BlockSpec((_O, _D), lambda i: (0, 0)),
            pl.BlockSpec((_O, 1), lambda i: (0, 0)),
            pl.BlockSpec((_O, _D + 2 * _O), lambda i: (0, 0)),
            pl.BlockSpec((_O, 1), lambda i: (0, 0)),
        ],
        out_specs=pl.BlockSpec((_BLK, _O), lambda i: (i, 0)),
        out_shape=jax.ShapeDtypeStruct((_B, _O), jnp.float32),
        scratch_shapes=[pltpu.VMEM((_R, _BLK), jnp.float32)],
    )(x, x, karr, W_opp1, b1, W_opp2, b2, W, br)
    return out


# speedup vs baseline: 1.0155x; 1.0155x over previous
import jax
import jax.numpy as jnp
import numpy as np
from jax import lax
from jax.experimental import pallas as pl
from jax.experimental.pallas import tpu as pltpu

_NS = 18
_B = 4096
_D = 256
_O = 6
_R = 2 * _O * _NS
_BLK = 1024

_ROT = ((13, 15, 26, 6), (17, 29, 16, 24))
_TINY = np.float32(np.finfo(np.float32).tiny)


def _key_cols():
    keys = jnp.concatenate(
        [jax.random.split(jax.random.fold_in(jax.random.key(42), op_i), _NS)
         for op_i in range(2)])
    kd = jax.random.key_data(keys).reshape(2, _NS, 2)
    k1 = jnp.broadcast_to(kd[:, None, :, 0], (2, _O, _NS)).reshape(_R)
    k2 = jnp.broadcast_to(kd[:, None, :, 1], (2, _O, _NS)).reshape(_R)
    cr = ((np.arange(_R) // _NS) % _O).astype(np.uint32)
    ks2 = k1 ^ k2 ^ np.uint32(0x1BD11BDA)
    return jnp.stack([k1, k2, ks2, ks2 + 1, k1 + 2, k2 + 3, ks2 + 4,
                      k1 + 5, k2 + cr], axis=1)


def _rotl(x, d):
    return lax.shift_left(x, np.uint32(d)) | lax.shift_right_logical(
        x, np.uint32(32 - d))


def _gen_gumbel(karr_ref, blk_base, blk, g_s):
    c = [karr_ref[:, j:j + 1] for j in range(9)]
    lanes = lax.broadcasted_iota(jnp.uint32, (1, blk), 1)
    x1 = c[8] + np.uint32(6) * (lanes + blk_base)
    x0 = jnp.broadcast_to(c[0], (_R, blk)) + jnp.zeros((_R, blk), jnp.uint32)
    inj = ((c[1], c[3]), (c[2], c[4]), (c[0], c[5]), (c[1], c[6]),
           (c[2], c[7]))
    for g in range(5):
        for r in _ROT[g % 2]:
            x0 = x0 + x1
            x1 = _rotl(x1, r)
            x1 = x1 ^ x0
        a, b = inj[g]
        x0 = x0 + a
        x1 = x1 + b
    bits = x0 ^ x1
    fb = lax.shift_right_logical(bits, np.uint32(9)) | np.uint32(0x3F800000)
    f = lax.bitcast_convert_type(fb, jnp.float32) - jnp.float32(1.0)
    u = lax.max(jnp.float32(_TINY), f + jnp.float32(_TINY))
    g_s[...] = -jnp.log(-jnp.log(u))


def _fwd_kernel(x_ref, xh_ref, karr_ref, w1_ref, b1_ref, w2_ref, b2_ref,
                w_ref, b_ref, out_ref, g_s):
    blk = x_ref.shape[0]
    blk_base = (pl.program_id(0) * blk).astype(jnp.uint32)
    _gen_gumbel(karr_ref, blk_base, blk, g_s)

    dn = (((1,), (1,)), ((), ()))
    wfull = w_ref[...]
    wx = wfull[:, :_D]
    xb = x_ref[...]
    xh = xh_ref[...]

    idxs = []
    tvecs = []
    for o, (wr, br) in enumerate(((w1_ref, b1_ref), (w2_ref, b2_ref))):
        wo = wr[...]
        bo = br[...]
        z = jax.lax.dot_general(wo, xb, dn,
                                preferred_element_type=jnp.float32) + bo
        m = jnp.max(z, axis=0, keepdims=True)
        e = jnp.exp(z - m)
        dist = e / jnp.sum(e, axis=0, keepdims=True)
        logits = jnp.log(dist)

        zh = jax.lax.dot_general(wo, xh, dn,
                                 preferred_element_type=jnp.float32) + bo
        mh = jnp.max(zh, axis=0, keepdims=True)
        eh = jnp.exp(zh - mh)
        disth = eh / jnp.sum(eh, axis=0, keepdims=True)
        tvecs.append(disth[0, :])

        best = None
        idx = None
        for cc in range(6):
            r = (o * 6 + cc) * _NS
            val = logits[cc:cc + 1, :] + g_s[r:r + _NS, :]
            if cc == 0:
                best = val
                idx = jnp.zeros_like(val)
            else:
                pred = val > best
                best = jnp.where(pred, val, best)
                idx = jnp.where(pred, jnp.float32(cc), idx)
        idxs.append(idx)

    y0 = jax.lax.dot_general(wx, xb, dn,
                             preferred_element_type=jnp.float32) + b_ref[...]

    accs = [jnp.broadcast_to(y0[j:j + 1, :], (_NS, blk)) for j in range(6)]
    ps = []
    for o in range(2):
        p = None
        for cc in range(6):
            mf = (idxs[o] == jnp.float32(cc)).astype(jnp.float32)
            tc = tvecs[o][cc]
            p = mf * tc if p is None else p + mf * tc
            for j in range(6):
                accs[j] = accs[j] + mf * wfull[j, _D + 6 * o + cc]
        ps.append(p)

    m = accs[0]
    for j in range(1, 6):
        m = jnp.maximum(m, accs[j])
    es = [jnp.exp(a - m) for a in accs]
    se = es[0]
    for j in range(1, 6):
        se = se + es[j]

    w = ps[0] * ps[1]
    u = w / se
    norm = jnp.sum(w, axis=0, keepdims=True)
    rows = [jnp.sum(u * es[j], axis=0, keepdims=True) / norm
            for j in range(6)]
    out_ref[...] = jnp.concatenate(rows, axis=0).T


def kernel(x, W_opp1, b_opp1, W_opp2, b_opp2, W, b):
    karr = _key_cols()
    b1 = b_opp1.reshape(_O, 1)
    b2 = b_opp2.reshape(_O, 1)
    br = b.reshape(_O, 1)

    out = pl.pallas_call(
        _fwd_kernel,
        grid=(_B // _BLK,),
        in_specs=[
            pl.BlockSpec((_BLK, _D), lambda i: (i, 0)),
            pl.BlockSpec((8, _D), lambda i: (0, 0)),
            pl.BlockSpec((_R, 9), lambda i: (0, 0)),
            pl.BlockSpec((_O, _D), lambda i: (0, 0)),
            pl.BlockSpec((_O, 1), lambda i: (0, 0)),
            pl.BlockSpec((_O, _D), lambda i: (0, 0)),
            pl.BlockSpec((_O, 1), lambda i: (0, 0)),
            pl.BlockSpec((_O, _D + 2 * _O), lambda i: (0, 0)),
            pl.BlockSpec((_O, 1), lambda i: (0, 0)),
        ],
        out_specs=pl.BlockSpec((_BLK, _O), lambda i: (i, 0)),
        out_shape=jax.ShapeDtypeStruct((_B, _O), jnp.float32),
        scratch_shapes=[pltpu.VMEM((_R, _BLK), jnp.float32)],
    )(x, x, karr, W_opp1, b1, W_opp2, b2, W, br)
    return out


# speedup vs baseline: 1.3350x; 1.3147x over previous
import jax
import jax.numpy as jnp
import numpy as np
from jax import lax
from jax.experimental import pallas as pl
from jax.experimental.pallas import tpu as pltpu

_NS = 18
_B = 4096
_D = 256
_O = 6
_R = 2 * _O * _NS
_BLK = 1024

_ROT = ((13, 15, 26, 6), (17, 29, 16, 24))
_TINY = np.float32(np.finfo(np.float32).tiny)


def _np_threefry(k1, k2, x0, x1):
    with np.errstate(over="ignore"):
        k1, k2 = np.uint32(k1), np.uint32(k2)
        x0 = np.asarray(x0, np.uint32)
        x1 = np.asarray(x1, np.uint32)
        ks = [k1, k2, np.uint32(k1 ^ k2 ^ np.uint32(0x1BD11BDA))]
        x0 = (x0 + ks[0]).astype(np.uint32)
        x1 = (x1 + ks[1]).astype(np.uint32)
        for g in range(5):
            for r in _ROT[g % 2]:
                x0 = (x0 + x1).astype(np.uint32)
                x1 = ((x1 << np.uint32(r))
                      | (x1 >> np.uint32(32 - r))).astype(np.uint32)
                x1 = x1 ^ x0
            x0 = (x0 + ks[(g + 1) % 3]).astype(np.uint32)
            x1 = (x1 + ks[(g + 2) % 3] + np.uint32(g + 1)).astype(np.uint32)
    return x0, x1


def _key_cols():
    kds = []
    for o in range(2):
        f0, f1 = _np_threefry(0, 42, np.uint32(0), np.uint32(o))
        b1, b2 = _np_threefry(f0, f1, np.zeros(_NS, np.uint32),
                              np.arange(_NS, dtype=np.uint32))
        kds.append(np.stack([b1, b2], 1))
    kd = np.stack(kds)
    k1 = np.broadcast_to(kd[:, None, :, 0], (2, _O, _NS)).reshape(_R)
    k2 = np.broadcast_to(kd[:, None, :, 1], (2, _O, _NS)).reshape(_R)
    cr = ((np.arange(_R) // _NS) % _O).astype(np.uint32)
    with np.errstate(over="ignore"):
        ks2 = (k1 ^ k2 ^ np.uint32(0x1BD11BDA)).astype(np.uint32)
        return np.stack(
            [k1, k2, ks2, ks2 + 1, k1 + 2, k2 + 3, ks2 + 4, k1 + 5,
             k2 + cr], axis=1).astype(np.uint32)


_KARR = _key_cols()


def _rotl(x, d):
    return lax.shift_left(x, np.uint32(d)) | lax.shift_right_logical(
        x, np.uint32(32 - d))


def _gen_gumbel(karr_ref, blk_base, blk, g_s):
    c = [karr_ref[:, j:j + 1] for j in range(9)]
    lanes = lax.broadcasted_iota(jnp.uint32, (1, blk), 1)
    x1 = c[8] + np.uint32(6) * (lanes + blk_base)
    x0 = jnp.broadcast_to(c[0], (_R, blk)) + jnp.zeros((_R, blk), jnp.uint32)
    inj = ((c[1], c[3]), (c[2], c[4]), (c[0], c[5]), (c[1], c[6]),
           (c[2], c[7]))
    for g in range(5):
        for r in _ROT[g % 2]:
            x0 = x0 + x1
            x1 = _rotl(x1, r)
            x1 = x1 ^ x0
        a, b = inj[g]
        x0 = x0 + a
        x1 = x1 + b
    bits = x0 ^ x1
    fb = lax.shift_right_logical(bits, np.uint32(9)) | np.uint32(0x3F800000)
    f = lax.bitcast_convert_type(fb, jnp.float32) - jnp.float32(1.0)
    u = lax.max(jnp.float32(_TINY), f + jnp.float32(_TINY))
    g_s[...] = -jnp.log(-jnp.log(u))


def _fwd_kernel(x_ref, xh_ref, karr_ref, w1_ref, b1_ref, w2_ref, b2_ref,
                w_ref, b_ref, out_ref, g_s):
    blk = x_ref.shape[0]
    blk_base = (pl.program_id(0) * blk).astype(jnp.uint32)
    _gen_gumbel(karr_ref, blk_base, blk, g_s)

    dn = (((1,), (1,)), ((), ()))
    wfull = w_ref[...]
    wx = wfull[:, :_D]
    xb = x_ref[...]
    xh = xh_ref[...]

    idxs = []
    tvecs = []
    for o, (wr, br) in enumerate(((w1_ref, b1_ref), (w2_ref, b2_ref))):
        wo = wr[...]
        bo = br[...]
        z = jax.lax.dot_general(wo, xb, dn,
                                preferred_element_type=jnp.float32) + bo
        m = jnp.max(z, axis=0, keepdims=True)
        e = jnp.exp(z - m)
        dist = e / jnp.sum(e, axis=0, keepdims=True)
        logits = jnp.log(dist)

        zh = jax.lax.dot_general(wo, xh, dn,
                                 preferred_element_type=jnp.float32) + bo
        mh = jnp.max(zh, axis=0, keepdims=True)
        eh = jnp.exp(zh - mh)
        disth = eh / jnp.sum(eh, axis=0, keepdims=True)
        tvecs.append(disth[0, :])

        best = None
        idx = None
        for cc in range(6):
            r = (o * 6 + cc) * _NS
            val = logits[cc:cc + 1, :] + g_s[r:r + _NS, :]
            if cc == 0:
                best = val
                idx = jnp.zeros_like(val)
            else:
                pred = val > best
                best = jnp.where(pred, val, best)
                idx = jnp.where(pred, jnp.float32(cc), idx)
        idxs.append(idx)

    y0 = jax.lax.dot_general(wx, xb, dn,
                             preferred_element_type=jnp.float32) + b_ref[...]

    accs = [jnp.broadcast_to(y0[j:j + 1, :], (_NS, blk)) for j in range(6)]
    ps = []
    for o in range(2):
        p = None
        for cc in range(6):
            mf = (idxs[o] == jnp.float32(cc)).astype(jnp.float32)
            tc = tvecs[o][cc]
            p = mf * tc if p is None else p + mf * tc
            for j in range(6):
                accs[j] = accs[j] + mf * wfull[j, _D + 6 * o + cc]
        ps.append(p)

    m = accs[0]
    for j in range(1, 6):
        m = jnp.maximum(m, accs[j])
    es = [jnp.exp(a - m) for a in accs]
    se = es[0]
    for j in range(1, 6):
        se = se + es[j]

    w = ps[0] * ps[1]
    u = w / se
    norm = jnp.sum(w, axis=0, keepdims=True)
    rows = [jnp.sum(u * es[j], axis=0, keepdims=True) / norm
            for j in range(6)]
    out_ref[...] = jnp.concatenate(rows, axis=0).T


def kernel(x, W_opp1, b_opp1, W_opp2, b_opp2, W, b):
    karr = _KARR
    b1 = b_opp1.reshape(_O, 1)
    b2 = b_opp2.reshape(_O, 1)
    br = b.reshape(_O, 1)

    out = pl.pallas_call(
        _fwd_kernel,
        grid=(_B // _BLK,),
        in_specs=[
            pl.BlockSpec((_BLK, _D), lambda i: (i, 0)),
            pl.BlockSpec((8, _D), lambda i: (0, 0)),
            pl.BlockSpec((_R, 9), lambda i: (0, 0)),
            pl.BlockSpec((_O, _D), lambda i: (0, 0)),
            pl.BlockSpec((_O, 1), lambda i: (0, 0)),
            pl.BlockSpec((_O, _D), lambda i: (0, 0)),
            pl.BlockSpec((_O, 1), lambda i: (0, 0)),
            pl.BlockSpec((_O, _D + 2 * _O), lambda i: (0, 0)),
            pl.BlockSpec((_O, 1), lambda i: (0, 0)),
        ],
        out_specs=pl.BlockSpec((_BLK, _O), lambda i: (i, 0)),
        out_shape=jax.ShapeDtypeStruct((_B, _O), jnp.float32),
        scratch_shapes=[pltpu.VMEM((_R, _BLK), jnp.float32)],
    )(x, x, karr, W_opp1, b1, W_opp2, b2, W, br)
    return out


# speedup vs baseline: 1.5141x; 1.1342x over previous
import jax
import jax.numpy as jnp
import numpy as np
from jax import lax
from jax.experimental import pallas as pl
from jax.experimental.pallas import tpu as pltpu

_NS = 18
_B = 4096
_D = 256
_O = 6
_R = 2 * _O * _NS
_BLK = 1024

_ROT = ((13, 15, 26, 6), (17, 29, 16, 24))
_TINY = np.float32(np.finfo(np.float32).tiny)


def _np_threefry(k1, k2, x0, x1):
    with np.errstate(over="ignore"):
        k1, k2 = np.uint32(k1), np.uint32(k2)
        x0 = np.asarray(x0, np.uint32)
        x1 = np.asarray(x1, np.uint32)
        ks = [k1, k2, np.uint32(k1 ^ k2 ^ np.uint32(0x1BD11BDA))]
        x0 = (x0 + ks[0]).astype(np.uint32)
        x1 = (x1 + ks[1]).astype(np.uint32)
        for g in range(5):
            for r in _ROT[g % 2]:
                x0 = (x0 + x1).astype(np.uint32)
                x1 = ((x1 << np.uint32(r))
                      | (x1 >> np.uint32(32 - r))).astype(np.uint32)
                x1 = x1 ^ x0
            x0 = (x0 + ks[(g + 1) % 3]).astype(np.uint32)
            x1 = (x1 + ks[(g + 2) % 3] + np.uint32(g + 1)).astype(np.uint32)
    return x0, x1


def _key_cols():
    kds = []
    for o in range(2):
        f0, f1 = _np_threefry(0, 42, np.uint32(0), np.uint32(o))
        b1, b2 = _np_threefry(f0, f1, np.zeros(_NS, np.uint32),
                              np.arange(_NS, dtype=np.uint32))
        kds.append(np.stack([b1, b2], 1))
    kd = np.stack(kds)
    k1 = np.broadcast_to(kd[:, None, :, 0], (2, _O, _NS)).reshape(_R)
    k2 = np.broadcast_to(kd[:, None, :, 1], (2, _O, _NS)).reshape(_R)
    cr = ((np.arange(_R) // _NS) % _O).astype(np.uint32)
    with np.errstate(over="ignore"):
        ks2 = (k1 ^ k2 ^ np.uint32(0x1BD11BDA)).astype(np.uint32)
        return np.stack(
            [k1, k2, ks2, ks2 + 1, k1 + 2, k2 + 3, ks2 + 4, k1 + 5,
             k2 + cr], axis=1).astype(np.uint32)


_KARR = _key_cols()


def _rotl(x, d):
    return lax.shift_left(x, np.uint32(d)) | lax.shift_right_logical(
        x, np.uint32(32 - d))


def _gen_gumbel(karr_ref, blk_base, blk, g_s):
    c = [karr_ref[:, j:j + 1] for j in range(9)]
    lanes = lax.broadcasted_iota(jnp.uint32, (1, blk), 1)
    x1 = c[8] + np.uint32(6) * (lanes + blk_base)
    x0 = jnp.broadcast_to(c[0], (_R, blk)) + jnp.zeros((_R, blk), jnp.uint32)
    inj = ((c[1], c[3]), (c[2], c[4]), (c[0], c[5]), (c[1], c[6]),
           (c[2], c[7]))
    for g in range(5):
        for r in _ROT[g % 2]:
            x0 = x0 + x1
            x1 = _rotl(x1, r)
            x1 = x1 ^ x0
        a, b = inj[g]
        x0 = x0 + a
        x1 = x1 + b
    bits = x0 ^ x1
    fb = lax.shift_right_logical(bits, np.uint32(9)) | np.uint32(0x3F800000)
    f = lax.bitcast_convert_type(fb, jnp.float32) - jnp.float32(1.0)
    u = lax.max(jnp.float32(_TINY), f + jnp.float32(_TINY))
    g_s[...] = -jnp.log(-jnp.log(u))


def _fwd_kernel(x_ref, karr_ref, w1_ref, b1_ref, w2_ref, b2_ref,
                w_ref, b_ref, out_ref, g_s, tv_s):
    blk = x_ref.shape[0]
    step = pl.program_id(0)
    blk_base = (step * blk).astype(jnp.uint32)
    _gen_gumbel(karr_ref, blk_base, blk, g_s)

    dn = (((1,), (1,)), ((), ()))
    wfull = w_ref[...]
    wx = wfull[:, :_D]
    xb = x_ref[...]

    def _col(br):
        bb = br[...]
        return jnp.concatenate([bb[0:1, cc:cc + 1] for cc in range(6)],
                               axis=0)

    bcols = [_col(b1_ref), _col(b2_ref)]
    zs = []
    logitss = []
    for o, wr in enumerate((w1_ref, w2_ref)):
        wo = wr[...]
        z = jax.lax.dot_general(wo, xb, dn,
                                preferred_element_type=jnp.float32) + bcols[o]
        m = jnp.max(z, axis=0, keepdims=True)
        e = jnp.exp(z - m)
        dist = e / jnp.sum(e, axis=0, keepdims=True)
        logitss.append(jnp.log(dist))
        zs.append(z)

    @pl.when(step == 0)
    def _():
        for o in range(2):
            zh = zs[o][:, 0:8]
            mh = jnp.max(zh, axis=0, keepdims=True)
            eh = jnp.exp(zh - mh)
            disth = eh / jnp.sum(eh, axis=0, keepdims=True)
            tv_s[o:o + 1, :] = disth[0:1, :]

    tvecs = [tv_s[0:1, :], tv_s[1:2, :]]

    idxs = []
    for o in range(2):
        logits = logitss[o]
        best = None
        idx = None
        for cc in range(6):
            r = (o * 6 + cc) * _NS
            val = logits[cc:cc + 1, :] + g_s[r:r + _NS, :]
            if cc == 0:
                best = val
                idx = jnp.zeros_like(val)
            else:
                pred = val > best
                best = jnp.where(pred, val, best)
                idx = jnp.where(pred, jnp.float32(cc), idx)
        idxs.append(idx)

    y0 = jax.lax.dot_general(wx, xb, dn,
                             preferred_element_type=jnp.float32) + _col(b_ref)

    accs = [jnp.broadcast_to(y0[j:j + 1, :], (_NS, blk)) for j in range(6)]
    ps = []
    for o in range(2):
        p = None
        for cc in range(6):
            mf = (idxs[o] == jnp.float32(cc)).astype(jnp.float32)
            tc = tvecs[o][0, cc]
            p = mf * tc if p is None else p + mf * tc
            for j in range(6):
                accs[j] = accs[j] + mf * wfull[j, _D + 6 * o + cc]
        ps.append(p)

    m = accs[0]
    for j in range(1, 6):
        m = jnp.maximum(m, accs[j])
    es = [jnp.exp(a - m) for a in accs]
    se = es[0]
    for j in range(1, 6):
        se = se + es[j]

    w = ps[0] * ps[1]
    u = w / se
    norm = jnp.sum(w, axis=0, keepdims=True)
    rows = [jnp.sum(u * es[j], axis=0, keepdims=True) / norm
            for j in range(6)]
    out_ref[...] = jnp.concatenate(rows, axis=0).T


def kernel(x, W_opp1, b_opp1, W_opp2, b_opp2, W, b):
    karr = _KARR
    b1 = b_opp1.reshape(1, _O)
    b2 = b_opp2.reshape(1, _O)
    br = b.reshape(1, _O)

    out = pl.pallas_call(
        _fwd_kernel,
        grid=(_B // _BLK,),
        in_specs=[
            pl.BlockSpec((_BLK, _D), lambda i: (i, 0)),
            pl.BlockSpec((_R, 9), lambda i: (0, 0)),
            pl.BlockSpec((_O, _D), lambda i: (0, 0)),
            pl.BlockSpec((1, _O), lambda i: (0, 0)),
            pl.BlockSpec((_O, _D), lambda i: (0, 0)),
            pl.BlockSpec((1, _O), lambda i: (0, 0)),
            pl.BlockSpec((_O, _D + 2 * _O), lambda i: (0, 0)),
            pl.BlockSpec((1, _O), lambda i: (0, 0)),
        ],
        out_specs=pl.BlockSpec((_BLK, _O), lambda i: (i, 0)),
        out_shape=jax.ShapeDtypeStruct((_B, _O), jnp.float32),
        scratch_shapes=[pltpu.VMEM((_R, _BLK), jnp.float32),
                        pltpu.VMEM((2, 8), jnp.float32)],
    )(x, karr, W_opp1, b1, W_opp2, b2, W, br)
    return out
